# Initial kernel scaffold; baseline (speedup 1.0000x reference)
import jax
import jax.numpy as jnp
from jax.experimental import pallas as pl


def kernel(tensor_1, tensor_2):
    raise NotImplementedError("write your pallas kernel here")


# speedup vs baseline: 3.9073x; 3.9073x over previous
import functools

import jax
import jax.numpy as jnp
from jax.experimental import pallas as pl
from jax.experimental.pallas import tpu as pltpu

_BM = 1024
_BN = 2048
_D = 128

_NEG = float("-inf")


def _insert(v, a, b, c):
    na = jnp.maximum(a, v)
    nb = jnp.maximum(b, jnp.minimum(a, v))
    nc = jnp.maximum(c, jnp.minimum(b, jnp.minimum(a, v)))
    return na, nb, nc


def _topk_kernel(q_ref, k_ref, o_ref, r1, r2, r3, *, n_keys, bn, nkb):
    j = pl.program_id(1)

    @pl.when(j == 0)
    def _init():
        r1[...] = jnp.full(r1.shape, _NEG, jnp.float32)
        r2[...] = jnp.full(r2.shape, _NEG, jnp.float32)
        r3[...] = jnp.full(r3.shape, _NEG, jnp.float32)

    q = q_ref[...]
    qn = jnp.sqrt(jnp.sum(q * q, axis=1, keepdims=True))
    qb = q / qn
    k = k_ref[...]
    kn = jnp.sqrt(jnp.sum(k * k, axis=1, keepdims=True))
    kb = k / kn

    s = jax.lax.dot_general(
        qb, kb, (((1,), (1,)), ((), ())), preferred_element_type=jnp.float32
    )
    col = jax.lax.broadcasted_iota(jnp.int32, s.shape, 1) + j * bn
    s = jnp.where(col < n_keys, s, _NEG)

    b1 = jnp.max(s, axis=1, keepdims=True)
    eq1 = s == b1
    c1 = jnp.sum(eq1.astype(jnp.float32), axis=1, keepdims=True)
    s2 = jnp.where(eq1, _NEG, s)
    b2 = jnp.max(s2, axis=1, keepdims=True)
    eq2 = s2 == b2
    c2 = jnp.sum(eq2.astype(jnp.float32), axis=1, keepdims=True)
    s3 = jnp.where(eq2, _NEG, s2)
    b3 = jnp.max(s3, axis=1, keepdims=True)

    t1 = b1
    t2 = jnp.where(c1 >= 2.0, b1, b2)
    t3 = jnp.where(
        c1 >= 3.0, b1, jnp.where(c1 == 2.0, b2, jnp.where(c2 >= 2.0, b2, b3))
    )

    a, b, c = r1[...], r2[...], r3[...]
    a, b, c = _insert(t1, a, b, c)
    a, b, c = _insert(t2, a, b, c)
    a, b, c = _insert(t3, a, b, c)
    r1[...], r2[...], r3[...] = a, b, c

    @pl.when(j == nkb - 1)
    def _done():
        o_ref[...] = (a + b + c) * jnp.float32(1.0 / 3.0)


def kernel(tensor_1, tensor_2):
    m, d = tensor_1.shape
    n_keys = tensor_2.shape[0]

    nkb = (n_keys + _BN - 1) // _BN
    n_pad = nkb * _BN
    if n_pad != n_keys:
        tensor_2 = jnp.pad(
            tensor_2, ((0, n_pad - n_keys), (0, 0)), constant_values=1.0
        )
    nqb = m // _BM

    out = pl.pallas_call(
        functools.partial(_topk_kernel, n_keys=n_keys, bn=_BN, nkb=nkb),
        grid=(nqb, nkb),
        in_specs=[
            pl.BlockSpec((_BM, d), lambda i, j: (i, 0)),
            pl.BlockSpec((_BN, d), lambda i, j: (j, 0)),
        ],
        out_specs=pl.BlockSpec((_BM, _D), lambda i, j: (i, 0)),
        out_shape=jax.ShapeDtypeStruct((m, _D), jnp.float32),
        scratch_shapes=[
            pltpu.VMEM((_BM, _D), jnp.float32),
            pltpu.VMEM((_BM, _D), jnp.float32),
            pltpu.VMEM((_BM, _D), jnp.float32),
        ],
        compiler_params=pltpu.CompilerParams(
            dimension_semantics=("parallel", "arbitrary"),
        ),
    )(tensor_1, tensor_2)
    return out[:, 0]


# speedup vs baseline: 6.8178x; 1.7449x over previous
import functools

import jax
import jax.numpy as jnp
from jax.experimental import pallas as pl
from jax.experimental.pallas import tpu as pltpu

_BM = 1024
_BN = 2048
_D = 128

_NEG = float("-inf")


def _topk_kernel(q_ref, k_ref, o_ref, r1, r2, r3, *, n_keys, bn, nkb):
    j = pl.program_id(1)

    @pl.when(j == 0)
    def _init():
        r1[...] = jnp.full(r1.shape, _NEG, jnp.float32)
        r2[...] = jnp.full(r2.shape, _NEG, jnp.float32)
        r3[...] = jnp.full(r3.shape, _NEG, jnp.float32)

    q = q_ref[...]
    qn = jnp.sqrt(jnp.sum(q * q, axis=1, keepdims=True))
    qb = q / qn
    k = k_ref[...]
    kn = jnp.sqrt(jnp.sum(k * k, axis=1, keepdims=True))
    kb = k / kn

    s = jax.lax.dot_general(
        qb, kb, (((1,), (1,)), ((), ())), preferred_element_type=jnp.float32
    )
    col = jax.lax.broadcasted_iota(jnp.int32, s.shape, 1) + j * bn
    s = jnp.where(col < n_keys, s, _NEG)

    a, b, c = r1[...], r2[...], r3[...]
    for ch in range(bn // _D):
        v = s[:, ch * _D:(ch + 1) * _D]
        m1 = jnp.minimum(a, v)
        a = jnp.maximum(a, v)
        m2 = jnp.minimum(b, m1)
        b = jnp.maximum(b, m1)
        c = jnp.maximum(c, m2)
    r1[...], r2[...], r3[...] = a, b, c

    @pl.when(j == nkb - 1)
    def _done():
        x = jnp.concatenate([a, b, c], axis=1)
        b1 = jnp.max(x, axis=1, keepdims=True)
        eq1 = x == b1
        c1 = jnp.sum(eq1.astype(jnp.float32), axis=1, keepdims=True)
        x2 = jnp.where(eq1, _NEG, x)
        b2 = jnp.max(x2, axis=1, keepdims=True)
        eq2 = x2 == b2
        c2 = jnp.sum(eq2.astype(jnp.float32), axis=1, keepdims=True)
        x3 = jnp.where(eq2, _NEG, x2)
        b3 = jnp.max(x3, axis=1, keepdims=True)

        t2 = jnp.where(c1 >= 2.0, b1, b2)
        t3 = jnp.where(
            c1 >= 3.0, b1, jnp.where(c1 == 2.0, b2, jnp.where(c2 >= 2.0, b2, b3))
        )
        mean = (b1 + t2 + t3) * jnp.float32(1.0 / 3.0)
        o_ref[...] = jnp.broadcast_to(mean, o_ref.shape)


def kernel(tensor_1, tensor_2):
    m, d = tensor_1.shape
    n_keys = tensor_2.shape[0]

    nkb = (n_keys + _BN - 1) // _BN
    n_pad = nkb * _BN
    if n_pad != n_keys:
        tensor_2 = jnp.pad(
            tensor_2, ((0, n_pad - n_keys), (0, 0)), constant_values=1.0
        )
    nqb = m // _BM

    out = pl.pallas_call(
        functools.partial(_topk_kernel, n_keys=n_keys, bn=_BN, nkb=nkb),
        grid=(nqb, nkb),
        in_specs=[
            pl.BlockSpec((_BM, d), lambda i, j: (i, 0)),
            pl.BlockSpec((_BN, d), lambda i, j: (j, 0)),
        ],
        out_specs=pl.BlockSpec((_BM, _D), lambda i, j: (i, 0)),
        out_shape=jax.ShapeDtypeStruct((m, _D), jnp.float32),
        scratch_shapes=[
            pltpu.VMEM((_BM, _D), jnp.float32),
            pltpu.VMEM((_BM, _D), jnp.float32),
            pltpu.VMEM((_BM, _D), jnp.float32),
        ],
        compiler_params=pltpu.CompilerParams(
            dimension_semantics=("parallel", "arbitrary"),
        ),
    )(tensor_1, tensor_2)
    return out[:, 0]
